# baseline (speedup 1.0000x reference)
import jax
import jax.numpy as jnp
from jax.experimental import pallas as pl
from jax.experimental.pallas import tpu as pltpu


def _gather_mean_body(idx_ref, feats_ref, mem_ref):
    mem_ref[...] = jnp.mean(feats_ref[...], axis=1, keepdims=True)


def _distance_body(feats_ref, mem_ref, pl_ref, out_ref):
    vblk, p, d = feats_ref.shape
    m = mem_ref.shape[0]
    x = feats_ref[...]
    mem = mem_ref[...]
    plab = pl_ref[...]
    f2 = jnp.sum(x * x, axis=2)
    m2 = jnp.sum(mem * mem, axis=1)
    xb = x.reshape(vblk * p, d).astype(jnp.bfloat16)
    mb = mem.astype(jnp.bfloat16)
    cross = jax.lax.dot_general(
        xb, mb, (((1,), (1,)), ((), ())),
        preferred_element_type=jnp.float32)
    t = (m2[None, :] - 2.0 * cross).reshape(vblk, p, m)
    s = jnp.min(t, axis=2)
    dist = jnp.sqrt(jnp.maximum(f2 + s, 0.0) * (1.0 / d))
    out_ref[...] = jnp.exp(-jnp.abs(dist - plab))


def kernel(extracted_features, pseudo_labels, hc_indices):
    V, ncrop, P, d = extracted_features.shape
    M = hc_indices.shape[0]
    VBLK = 8
    feats = extracted_features.reshape(V, P, d)
    idx = hc_indices.astype(jnp.int32)

    memory = pl.pallas_call(
        _gather_mean_body,
        grid_spec=pltpu.PrefetchScalarGridSpec(
            num_scalar_prefetch=1,
            grid=(M,),
            in_specs=[pl.BlockSpec((1, P, d), lambda i, idx_ref: (idx_ref[i], 0, 0))],
            out_specs=pl.BlockSpec((1, 1, d), lambda i, idx_ref: (i, 0, 0)),
        ),
        out_shape=jax.ShapeDtypeStruct((M, 1, d), jnp.float32),
    )(idx, feats).reshape(M, d)

    out = pl.pallas_call(
        _distance_body,
        grid=(V // VBLK,),
        in_specs=[
            pl.BlockSpec((VBLK, P, d), lambda i: (i, 0, 0)),
            pl.BlockSpec((M, d), lambda i: (0, 0)),
            pl.BlockSpec((VBLK, P), lambda i: (i, 0)),
        ],
        out_specs=pl.BlockSpec((VBLK, P), lambda i: (i, 0)),
        out_shape=jax.ShapeDtypeStruct((V, P), jnp.float32),
    )(feats, memory, pseudo_labels)
    return out


# speedup vs baseline: 22.3949x; 22.3949x over previous
import jax
import jax.numpy as jnp
from jax.experimental import pallas as pl
from jax.experimental.pallas import tpu as pltpu

_GBLK = 16


def _gather_mean_body(idx_ref, *refs):
    feat_refs, mem_ref = refs[:-1], refs[-1]
    means = [jnp.mean(f[...], axis=1, keepdims=True) for f in feat_refs]
    mem_ref[...] = jnp.concatenate(means, axis=0)


def _distance_body(feats_ref, mem_ref, pl_ref, out_ref):
    vblk, p, d = feats_ref.shape
    m = mem_ref.shape[0]
    r = vblk * p
    x = feats_ref[...].reshape(r, d)
    xb = x.astype(jnp.bfloat16)
    xb2 = xb * xb
    mem = mem_ref[...]
    mb = mem.astype(jnp.bfloat16)
    mb2 = mb * mb
    ones8 = jnp.ones((8, d), jnp.bfloat16)
    crossT = jax.lax.dot_general(
        mb, xb, (((1,), (1,)), ((), ())),
        preferred_element_type=jnp.float32)
    f2r = jax.lax.dot_general(
        ones8, xb2, (((1,), (1,)), ((), ())),
        preferred_element_type=jnp.float32)
    m2c = jax.lax.dot_general(
        mb2, ones8, (((1,), (1,)), ((), ())),
        preferred_element_type=jnp.float32)
    t = m2c[:, 0:1] - 2.0 * crossT
    s8 = t[0:8]
    for k in range(1, m // 8):
        s8 = jnp.minimum(s8, t[8 * k:8 * k + 8])
    s1 = jnp.min(s8, axis=0, keepdims=True)
    sq = jnp.maximum(f2r[0:1] + s1, 0.0)
    dist = jnp.sqrt(sq * (1.0 / d))
    out_ref[...] = jnp.exp(-jnp.abs(dist[None] - pl_ref[...]))


def kernel(extracted_features, pseudo_labels, hc_indices):
    V, ncrop, P, d = extracted_features.shape
    M = hc_indices.shape[0]
    VBLK = 32
    feats = extracted_features.reshape(V, P, d)
    idx = hc_indices.astype(jnp.int32)

    gather_specs = [
        pl.BlockSpec((1, P, d),
                     lambda i, idx_ref, k=k: (idx_ref[_GBLK * i + k], 0, 0))
        for k in range(_GBLK)
    ]
    memory = pl.pallas_call(
        _gather_mean_body,
        grid_spec=pltpu.PrefetchScalarGridSpec(
            num_scalar_prefetch=1,
            grid=(M // _GBLK,),
            in_specs=gather_specs,
            out_specs=pl.BlockSpec((_GBLK, 1, d), lambda i, idx_ref: (i, 0, 0)),
        ),
        out_shape=jax.ShapeDtypeStruct((M, 1, d), jnp.float32),
    )(idx, *([feats] * _GBLK)).reshape(M, d)

    R = VBLK * P
    plab3 = pseudo_labels.reshape(V // VBLK, 1, R)
    out = pl.pallas_call(
        _distance_body,
        grid=(V // VBLK,),
        in_specs=[
            pl.BlockSpec((VBLK, P, d), lambda i: (i, 0, 0)),
            pl.BlockSpec((M, d), lambda i: (0, 0)),
            pl.BlockSpec((1, 1, R), lambda i: (i, 0, 0)),
        ],
        out_specs=pl.BlockSpec((1, 1, R), lambda i: (i, 0, 0)),
        out_shape=jax.ShapeDtypeStruct((V // VBLK, 1, R), jnp.float32),
    )(feats, memory, plab3)
    return out.reshape(V, P)
